# baseline (speedup 1.0000x reference)
import functools

import jax
import jax.numpy as jnp
from jax import lax
from jax.experimental import pallas as pl
from jax.experimental.pallas import tpu as pltpu
from jax.experimental.pallas import tpu_sc as plsc

_SLOPE = 5.0
_BUDGET = 1.0 / 16.0 - 1.0 / 128.0
_RATIO = 128
_C_LO = 146
_C_HI = 174
_H = 320
_W = 320
_HW = _H * _W

_KS0 = 0
_KS1 = 42
_KS2 = 0x1BD11BDA ^ _KS0 ^ _KS1

_RC = 16
_NCHUNK = _H // _RC
_LANE = 16
_VPR = _W // _LANE


def _rotl(x, d):
    return (x << jnp.uint32(d)) | (x >> jnp.uint32(32 - d))


def _threefry2x32_zero_x0(x1_plus_ks1):
    ks0 = jnp.uint32(_KS0)
    ks1 = jnp.uint32(_KS1)
    ks2 = jnp.uint32(_KS2)

    def four_rounds(x0, x1, rots):
        for r in rots:
            x0 = x0 + x1
            x1 = _rotl(x1, r) ^ x0
        return x0, x1

    x0 = x1_plus_ks1
    x1 = _rotl(x1_plus_ks1, 13) ^ x0
    x0, x1 = four_rounds(x0, x1, (15, 26, 6))
    x0 = x0 + ks1
    x1 = x1 + jnp.uint32(_KS2 + 1)
    x0, x1 = four_rounds(x0, x1, (17, 29, 16, 24))
    x0 = x0 + ks2
    x1 = x1 + jnp.uint32(_KS0 + 2)
    x0, x1 = four_rounds(x0, x1, (13, 15, 26, 6))
    x0 = x0 + ks0
    x1 = x1 + jnp.uint32(_KS1 + 3)
    x0, x1 = four_rounds(x0, x1, (17, 29, 16, 24))
    x0 = x0 + ks1
    x1 = x1 + jnp.uint32(_KS2 + 4)
    x0, x1 = four_rounds(x0, x1, (13, 15, 26, 6))
    x0 = x0 + ks2
    x1 = x1 + jnp.uint32(_KS0 + 5)
    return x0, x1


def _prep_body(w_ref, thr_ref):
    row = lax.broadcasted_iota(jnp.int32, (_H, _W), 0)
    col = lax.broadcasted_iota(jnp.int32, (_H, _W), 1)
    prob = jax.nn.sigmoid(jnp.float32(_SLOPE) * w_ref[...])
    inside = (row >= _C_LO) & (row < _C_HI) & (col >= _C_LO) & (col < _C_HI)
    prob = jnp.where(inside, jnp.float32(0.0), prob)
    xbar = jnp.mean(prob)
    r = jnp.float32(_BUDGET) / xbar
    beta = (jnp.float32(1.0) - jnp.float32(_BUDGET)) / (jnp.float32(1.0) - xbar)
    mr = jnp.where(
        r <= jnp.float32(1.0),
        prob * r,
        jnp.float32(1.0) - (jnp.float32(1.0) - prob) * beta,
    )
    thr_ref[...] = jnp.ceil(mr * jnp.float32(8388608.0)).astype(jnp.int32)


def _sc_body(thr_hbm, ks_hbm, oks_hbm, mask_hbm, thr_v, mask_v, ks_v, out_v):
    wid = lax.axis_index("s") * 2 + lax.axis_index("c")
    lane = lax.iota(jnp.int32, _LANE)

    def per_batch(bk, _):
        gb = wid * 2 + bk

        def per_chunk(chunk, _):
            row0 = chunk * _RC
            pltpu.sync_copy(thr_hbm.at[pl.ds(row0, _RC)], thr_v)

            def per_row(r, _):
                base = (gb * _HW + (row0 + r) * _W) + _KS1

                def per_cgroup(c5, _):
                    for u in range(5):
                        c = c5 * 5 + u
                        x1 = lax.convert_element_type(
                            lane + (base + c * _LANE), jnp.uint32
                        )
                        o0, o1 = _threefry2x32_zero_x0(x1)
                        mant = lax.convert_element_type(
                            (o0 ^ o1) >> jnp.uint32(9), jnp.int32
                        )
                        thr_vec = thr_v[r, pl.ds(c * _LANE, _LANE)]
                        m = jnp.where(
                            mant < thr_vec, jnp.float32(1.0), jnp.float32(0.0)
                        )
                        mask_v[r, pl.ds(c * _LANE, _LANE)] = m
                    return _

                return lax.fori_loop(0, _VPR // 5, per_cgroup, _)

            lax.fori_loop(0, _RC, per_row, None)
            pltpu.sync_copy(mask_v, mask_hbm.at[gb, pl.ds(row0, _RC)])

            for ch in range(2):
                pltpu.sync_copy(ks_hbm.at[gb, ch, pl.ds(row0, _RC)], ks_v)

                def per_row_mul(r, _):
                    def per_cgroup_mul(c5, _):
                        for u in range(5):
                            c = c5 * 5 + u
                            sl = pl.ds(c * _LANE, _LANE)
                            out_v[r, sl] = ks_v[r, sl] * mask_v[r, sl]
                        return _

                    return lax.fori_loop(0, _VPR // 5, per_cgroup_mul, _)

                lax.fori_loop(0, _RC, per_row_mul, None)
                pltpu.sync_copy(out_v, oks_hbm.at[gb, ch, pl.ds(row0, _RC)])
            return _

        return lax.fori_loop(0, _NCHUNK, per_chunk, _)

    lax.fori_loop(0, 2, per_batch, None)


@jax.jit
def kernel(kspace, weight):
    B, C = kspace.shape[0], kspace.shape[1]
    thr = pl.pallas_call(
        _prep_body,
        out_shape=jax.ShapeDtypeStruct((_H, _W), jnp.int32),
    )(weight)

    mesh = plsc.VectorSubcoreMesh(core_axis_name="c", subcore_axis_name="s")
    sc = functools.partial(
        pl.kernel,
        mesh=mesh,
        out_type=[
            jax.ShapeDtypeStruct((B, C, _H, _W), jnp.float32),
            jax.ShapeDtypeStruct((B, _H, _W), jnp.float32),
        ],
        scratch_types=[
            pltpu.VMEM((_RC, _W), jnp.int32),
            pltpu.VMEM((_RC, _W), jnp.float32),
            pltpu.VMEM((_RC, _W), jnp.float32),
            pltpu.VMEM((_RC, _W), jnp.float32),
        ],
    )(_sc_body)
    oks, mask = sc(thr, kspace)
    return (oks, mask, jnp.asarray(_RATIO, dtype=jnp.int32))


# speedup vs baseline: 3.8444x; 3.8444x over previous
import jax
import jax.numpy as jnp
from jax import lax
from jax.experimental import pallas as pl
from jax.experimental.pallas import tpu as pltpu

_SLOPE = 5.0
_BUDGET = 1.0 / 16.0 - 1.0 / 128.0
_RATIO = 128
_C_LO = 146
_C_HI = 174
_H = 320
_W = 320
_HW = _H * _W
_BPB = 4

_KS0 = 0
_KS1 = 42
_KS2 = 0x1BD11BDA ^ _KS0 ^ _KS1


def _rotl(x, d):
    return (x << jnp.uint32(d)) | (x >> jnp.uint32(32 - d))


def _threefry2x32_zero_x0(x1_plus_ks1):
    ks0 = jnp.uint32(_KS0)
    ks1 = jnp.uint32(_KS1)
    ks2 = jnp.uint32(_KS2)
    r_a = (13, 15, 26, 6)
    r_b = (17, 29, 16, 24)

    def four_rounds(x0, x1, rots):
        for r in rots:
            x0 = x0 + x1
            x1 = _rotl(x1, r) ^ x0
        return x0, x1

    x0 = x1_plus_ks1
    x1 = _rotl(x1_plus_ks1, 13) ^ x0
    x0, x1 = four_rounds(x0, x1, (15, 26, 6))
    x0 = x0 + ks1
    x1 = x1 + jnp.uint32(_KS2 + 1)
    x0, x1 = four_rounds(x0, x1, r_b)
    x0 = x0 + ks2
    x1 = x1 + jnp.uint32(_KS0 + 2)
    x0, x1 = four_rounds(x0, x1, r_a)
    x0 = x0 + ks0
    x1 = x1 + jnp.uint32(_KS1 + 3)
    x0, x1 = four_rounds(x0, x1, r_b)
    x0 = x0 + ks1
    x1 = x1 + jnp.uint32(_KS2 + 4)
    x0, x1 = four_rounds(x0, x1, r_a)
    x0 = x0 + ks2
    x1 = x1 + jnp.uint32(_KS0 + 5)
    return x0, x1


def _body(w_ref, ks_ref, oks_ref, mask_ref, thr_ref, ju_ref):
    b = pl.program_id(0)

    @pl.when(b == 0)
    def _prep():
        row = lax.broadcasted_iota(jnp.int32, (_H, _W), 0)
        col = lax.broadcasted_iota(jnp.int32, (_H, _W), 1)
        prob = jax.nn.sigmoid(jnp.float32(_SLOPE) * w_ref[...])
        inside = (row >= _C_LO) & (row < _C_HI) & (col >= _C_LO) & (col < _C_HI)
        prob = jnp.where(inside, jnp.float32(0.0), prob)
        xbar = jnp.mean(prob)
        r = jnp.float32(_BUDGET) / xbar
        beta = (jnp.float32(1.0) - jnp.float32(_BUDGET)) / (jnp.float32(1.0) - xbar)
        mr = jnp.where(
            r <= jnp.float32(1.0),
            prob * r,
            jnp.float32(1.0) - (jnp.float32(1.0) - prob) * beta,
        )
        thr_ref[...] = jnp.ceil(mr * jnp.float32(8388608.0)).astype(jnp.uint32)
        ju_ref[...] = (row * _W + col).astype(jnp.uint32) + jnp.uint32(_KS1)

    ju = ju_ref[...]
    thr = thr_ref[...]
    for bi in range(_BPB):
        x1 = ju + lax.convert_element_type((b * _BPB + bi) * _HW, jnp.uint32)
        o0, o1 = _threefry2x32_zero_x0(x1)
        mant = (o0 ^ o1) >> jnp.uint32(9)
        m = (mant < thr).astype(jnp.float32)
        mask_ref[bi] = m
        oks_ref[bi] = ks_ref[bi] * m[None]


@jax.jit
def kernel(kspace, weight):
    B, C = kspace.shape[0], kspace.shape[1]
    oks, mask = pl.pallas_call(
        _body,
        grid=(B // _BPB,),
        in_specs=[
            pl.BlockSpec((_H, _W), lambda b: (0, 0)),
            pl.BlockSpec((_BPB, C, _H, _W), lambda b: (b, 0, 0, 0)),
        ],
        out_specs=[
            pl.BlockSpec((_BPB, C, _H, _W), lambda b: (b, 0, 0, 0)),
            pl.BlockSpec((_BPB, _H, _W), lambda b: (b, 0, 0)),
        ],
        out_shape=[
            jax.ShapeDtypeStruct((B, C, _H, _W), jnp.float32),
            jax.ShapeDtypeStruct((B, _H, _W), jnp.float32),
        ],
        scratch_shapes=[
            pltpu.VMEM((_H, _W), jnp.uint32),
            pltpu.VMEM((_H, _W), jnp.uint32),
        ],
    )(weight, kspace)
    return (
        oks,
        mask,
        jnp.asarray(_RATIO, dtype=jnp.int32),
    )


# speedup vs baseline: 3.8756x; 1.0081x over previous
import jax
import jax.numpy as jnp
from jax import lax
from jax.experimental import pallas as pl
from jax.experimental.pallas import tpu as pltpu

_SLOPE = 5.0
_BUDGET = 1.0 / 16.0 - 1.0 / 128.0
_RATIO = 128
_C_LO = 146
_C_HI = 174
_H = 320
_W = 320
_HW = _H * _W
_BPB = 2

_KS0 = 0
_KS1 = 42
_KS2 = 0x1BD11BDA ^ _KS0 ^ _KS1


def _rotl(x, d):
    return (x << jnp.uint32(d)) | (x >> jnp.uint32(32 - d))


def _threefry2x32_zero_x0(x1_plus_ks1):
    ks0 = jnp.uint32(_KS0)
    ks1 = jnp.uint32(_KS1)
    ks2 = jnp.uint32(_KS2)
    r_a = (13, 15, 26, 6)
    r_b = (17, 29, 16, 24)

    def four_rounds(x0, x1, rots):
        for r in rots:
            x0 = x0 + x1
            x1 = _rotl(x1, r) ^ x0
        return x0, x1

    x0 = x1_plus_ks1
    x1 = _rotl(x1_plus_ks1, 13) ^ x0
    x0, x1 = four_rounds(x0, x1, (15, 26, 6))
    x0 = x0 + ks1
    x1 = x1 + jnp.uint32(_KS2 + 1)
    x0, x1 = four_rounds(x0, x1, r_b)
    x0 = x0 + ks2
    x1 = x1 + jnp.uint32(_KS0 + 2)
    x0, x1 = four_rounds(x0, x1, r_a)
    x0 = x0 + ks0
    x1 = x1 + jnp.uint32(_KS1 + 3)
    x0, x1 = four_rounds(x0, x1, r_b)
    x0 = x0 + ks1
    x1 = x1 + jnp.uint32(_KS2 + 4)
    x0, x1 = four_rounds(x0, x1, r_a)
    x0 = x0 + ks2
    x1 = x1 + jnp.uint32(_KS0 + 5)
    return x0, x1


def _body(w_ref, ks_ref, oks_ref, mask_ref, thr_ref, ju_ref):
    b = pl.program_id(0)

    @pl.when(b == 0)
    def _prep():
        row = lax.broadcasted_iota(jnp.int32, (_H, _W), 0)
        col = lax.broadcasted_iota(jnp.int32, (_H, _W), 1)
        prob = jax.nn.sigmoid(jnp.float32(_SLOPE) * w_ref[...])
        inside = (row >= _C_LO) & (row < _C_HI) & (col >= _C_LO) & (col < _C_HI)
        prob = jnp.where(inside, jnp.float32(0.0), prob)
        xbar = jnp.mean(prob)
        r = jnp.float32(_BUDGET) / xbar
        beta = (jnp.float32(1.0) - jnp.float32(_BUDGET)) / (jnp.float32(1.0) - xbar)
        mr = jnp.where(
            r <= jnp.float32(1.0),
            prob * r,
            jnp.float32(1.0) - (jnp.float32(1.0) - prob) * beta,
        )
        thr_ref[...] = jnp.ceil(mr * jnp.float32(8388608.0)).astype(jnp.uint32)
        ju_ref[...] = (row * _W + col).astype(jnp.uint32) + jnp.uint32(_KS1)

    ju = ju_ref[...]
    thr = thr_ref[...]
    for bi in range(_BPB):
        x1 = ju + lax.convert_element_type((b * _BPB + bi) * _HW, jnp.uint32)
        o0, o1 = _threefry2x32_zero_x0(x1)
        mant = (o0 ^ o1) >> jnp.uint32(9)
        m = (mant < thr).astype(jnp.float32)
        mask_ref[bi] = m
        oks_ref[bi] = ks_ref[bi] * m[None]


@jax.jit
def kernel(kspace, weight):
    B, C = kspace.shape[0], kspace.shape[1]
    oks, mask = pl.pallas_call(
        _body,
        grid=(B // _BPB,),
        in_specs=[
            pl.BlockSpec((_H, _W), lambda b: (0, 0)),
            pl.BlockSpec((_BPB, C, _H, _W), lambda b: (b, 0, 0, 0)),
        ],
        out_specs=[
            pl.BlockSpec((_BPB, C, _H, _W), lambda b: (b, 0, 0, 0)),
            pl.BlockSpec((_BPB, _H, _W), lambda b: (b, 0, 0)),
        ],
        out_shape=[
            jax.ShapeDtypeStruct((B, C, _H, _W), jnp.float32),
            jax.ShapeDtypeStruct((B, _H, _W), jnp.float32),
        ],
        scratch_shapes=[
            pltpu.VMEM((_H, _W), jnp.uint32),
            pltpu.VMEM((_H, _W), jnp.uint32),
        ],
    )(weight, kspace)
    return (
        oks,
        mask,
        jnp.asarray(_RATIO, dtype=jnp.int32),
    )
